# Initial kernel scaffold; baseline (speedup 1.0000x reference)
import jax
import jax.numpy as jnp
from jax.experimental import pallas as pl


def kernel(x, edge_index, W1, al1, ar1, W2, al2, ar2):
    raise NotImplementedError("write your pallas kernel here")


# speedup vs baseline: 9.1563x; 9.1563x over previous
import dataclasses
import functools

import jax
import jax.numpy as jnp
from jax import lax
from jax.experimental import pallas as pl
from jax.experimental.pallas import tpu as pltpu
from jax.experimental.pallas import tpu_sc as plsc

N = 10000
E = 320000
D = 128
NEG = 0.2

NW = 32
NB = 320
NA = 10496
NE_T = 10016
C = 1280
NCH = E // C
G = C // 16
W = 32

_CP = pltpu.CompilerParams()
if "needs_layout_passes" in pltpu.CompilerParams.__dataclass_fields__:
    _CP = dataclasses.replace(_CP, needs_layout_passes=False)

_MESH = plsc.VectorSubcoreMesh(core_axis_name="c", subcore_axis_name="s")


def _tc_layer1(xp, W1, al1, ar1):
    BR = 256
    grid = (NA // BR,)

    def body(x_ref, w_ref, al_ref, ar_ref, fA, fB, elA, erA, elB, erB):
        feat = jnp.dot(x_ref[...], w_ref[...], preferred_element_type=jnp.float32)
        fA[...] = feat[:, :256]
        fB[...] = feat[:, 256:]
        f4 = feat.reshape(BR, 4, 128)
        el = jnp.sum(f4 * al_ref[...][None], axis=-1)
        er = jnp.sum(f4 * ar_ref[...][None], axis=-1)
        elA[...] = el[:, 0:2]
        erA[...] = er[:, 0:2]
        elB[...] = el[:, 2:4]
        erB[...] = er[:, 2:4]

    f32 = jnp.float32
    return pl.pallas_call(
        body,
        grid=grid,
        in_specs=[
            pl.BlockSpec((BR, D), lambda i: (i, 0)),
            pl.BlockSpec((D, 512), lambda i: (0, 0)),
            pl.BlockSpec((4, 128), lambda i: (0, 0)),
            pl.BlockSpec((4, 128), lambda i: (0, 0)),
        ],
        out_specs=[
            pl.BlockSpec((BR, 256), lambda i: (i, 0)),
            pl.BlockSpec((BR, 256), lambda i: (i, 0)),
            pl.BlockSpec((BR, 2), lambda i: (i, 0)),
            pl.BlockSpec((BR, 2), lambda i: (i, 0)),
            pl.BlockSpec((BR, 2), lambda i: (i, 0)),
            pl.BlockSpec((BR, 2), lambda i: (i, 0)),
        ],
        out_shape=[
            jax.ShapeDtypeStruct((NA, 256), f32),
            jax.ShapeDtypeStruct((NA, 256), f32),
            jax.ShapeDtypeStruct((NA, 2), f32),
            jax.ShapeDtypeStruct((NA, 2), f32),
            jax.ShapeDtypeStruct((NA, 2), f32),
            jax.ShapeDtypeStruct((NA, 2), f32),
        ],
    )(xp, W1, al1, ar1)


def _tc_layer2(o1A, o1B, W2, al2, ar2):
    BR = 256
    grid = (NA // BR,)

    def body(a_ref, b_ref, w_ref, al_ref, ar_ref, f2, el2, er2):
        a = a_ref[...]
        b = b_ref[...]

        def elu(v):
            return jnp.where(v > 0, v, jnp.exp(v) - 1.0)

        hmean = 0.25 * (elu(a[:, :128]) + elu(a[:, 128:])
                        + elu(b[:, :128]) + elu(b[:, 128:]))
        f = jnp.dot(hmean, w_ref[...], preferred_element_type=jnp.float32)
        f2[...] = f
        el2[...] = jnp.sum(f * al_ref[...], axis=-1, keepdims=True)
        er2[...] = jnp.sum(f * ar_ref[...], axis=-1, keepdims=True)

    f32 = jnp.float32
    return pl.pallas_call(
        body,
        grid=grid,
        in_specs=[
            pl.BlockSpec((BR, 256), lambda i: (i, 0)),
            pl.BlockSpec((BR, 256), lambda i: (i, 0)),
            pl.BlockSpec((128, 128), lambda i: (0, 0)),
            pl.BlockSpec((1, 128), lambda i: (0, 0)),
            pl.BlockSpec((1, 128), lambda i: (0, 0)),
        ],
        out_specs=[
            pl.BlockSpec((BR, 128), lambda i: (i, 0)),
            pl.BlockSpec((BR, 1), lambda i: (i, 0)),
            pl.BlockSpec((BR, 1), lambda i: (i, 0)),
        ],
        out_shape=[
            jax.ShapeDtypeStruct((NA, 128), f32),
            jax.ShapeDtypeStruct((NA, 1), f32),
            jax.ShapeDtypeStruct((NA, 1), f32),
        ],
    )(o1A, o1B, W2, al2, ar2)


def _sc_pass(TW, HH, edge, tab, el, er):
    CPH = 128 // 16
    NC_ = TW // 16
    WG = W // 16

    @functools.partial(
        pl.kernel,
        out_type=jax.ShapeDtypeStruct((NA, TW), jnp.float32),
        mesh=_MESH,
        scratch_types=[
            pltpu.VMEM((NB + 1, TW), jnp.float32),
            pltpu.VMEM((NE_T * HH,), jnp.float32),
            pltpu.VMEM((336 * HH,), jnp.float32),
            pltpu.VMEM((352 * HH,), jnp.float32),
            pltpu.VMEM((C,), jnp.int32),
            pltpu.VMEM((C,), jnp.int32),
            pltpu.VMEM((C,), jnp.int32),
            pltpu.VMEM((C,), jnp.int32),
            pltpu.VMEM((C + 64,), jnp.int32),
            pltpu.VMEM((C + 64,), jnp.int32),
            pltpu.VMEM((W, TW), jnp.float32),
            pltpu.VMEM((96,), jnp.float32),
            pltpu.SemaphoreType.DMA,
            pltpu.SemaphoreType.DMA,
            pltpu.SemaphoreType.DMA,
        ],
        compiler_params=_CP,
    )
    def kern(edge_h, tab_h, el_h, er_h, out_h,
             acc, el_v, er_v, den_v, srcA, dstA, srcB, dstB, csrc, cdst,
             rows, eebuf, semg, semA, semB):
        cid = lax.axis_index("c")
        sid = lax.axis_index("s")
        wid = sid * 2 + cid
        base = wid * NB
        z16 = jnp.zeros((16,), jnp.float32)

        for q in range(352 * HH // 16):
            den_v[pl.ds(16 * q, 16)] = z16

        def zr(j, _):
            for cc in range(NC_):
                acc[j, pl.ds(16 * cc, 16)] = z16
            return 0
        lax.fori_loop(0, NB + 1, zr, 0)

        pltpu.sync_copy(el_h.at[pl.ds(0, NE_T * HH)], el_v)
        pltpu.sync_copy(er_h.at[pl.ds(base * HH, 336 * HH)], er_v)

        def do_wave(off):
            gcp = pltpu.async_copy(tab_h.at[csrc.at[pl.ds(off, W)]], rows, semg)
            for h in range(HH):
                for q in range(WG):
                    sv = csrc[pl.ds(off + 16 * q, 16)]
                    rl = cdst[pl.ds(off + 16 * q, 16)]
                    elh = plsc.load_gather(el_v, [sv * HH + h])
                    erh = plsc.load_gather(er_v, [rl * HH + h])
                    e = elh + erh
                    e = jnp.maximum(e, NEG * e)
                    ee = jnp.exp(e)
                    plsc.addupdate_scatter(den_v, [rl * HH + h], ee)
                    eebuf[pl.ds(h * W + 16 * q, 16)] = ee
            gcp.wait()

            def accj(j, _):
                rl = cdst[pl.ds(off + j, 16)][0]
                for h in range(HH):
                    eesc = eebuf[pl.ds(h * W + j, 16)][0]
                    for cc in range(CPH):
                        c = h * CPH + cc
                        val = rows[j, pl.ds(16 * c, 16)] * eesc
                        plsc.addupdate(acc.at[rl, pl.ds(16 * c, 16)], val)
                return 0
            lax.fori_loop(0, W, accj, 0)

        def scan_chunk(sbuf, dbuf, fill):
            def grp(g, fl):
                dvec = dbuf[pl.ds(16 * g, 16)]
                svec = sbuf[pl.ds(16 * g, 16)]
                rel = dvec - base
                mask = rel.astype(jnp.uint32) < jnp.uint32(NB)
                plsc.store_compressed(csrc.at[pl.ds(fl, 16)], svec, mask=mask)
                plsc.store_compressed(cdst.at[pl.ds(fl, 16)], rel, mask=mask)
                cnt = plsc.all_reduce_population_count(mask)
                return fl + (cnt if cnt.ndim == 0 else cnt[0])

            fill = lax.fori_loop(0, G, grp, fill)
            nw = fill // W

            def wv(w, _):
                do_wave(w * W)
                return 0
            lax.fori_loop(0, nw, wv, 0)
            tail = nw * W
            for q in range(WG):
                sv = csrc[pl.ds(tail + 16 * q, 16)]
                rv = cdst[pl.ds(tail + 16 * q, 16)]
                csrc[pl.ds(16 * q, 16)] = sv
                cdst[pl.ds(16 * q, 16)] = rv
            return fill - tail

        def stage(k, sbuf, dbuf, sem):
            pltpu.async_copy(edge_h.at[0, pl.ds(k * C, C)], sbuf, sem)
            pltpu.async_copy(edge_h.at[1, pl.ds(k * C, C)], dbuf, sem)

        def wait_stage(sbuf, dbuf, sem):
            pltpu.make_async_copy(edge_h.at[0, pl.ds(0, C)], sbuf, sem).wait()
            pltpu.make_async_copy(edge_h.at[1, pl.ds(0, C)], dbuf, sem).wait()

        stage(0, srcA, dstA, semA)

        def pair(i, fill):
            k0 = 2 * i
            stage(k0 + 1, srcB, dstB, semB)
            wait_stage(srcA, dstA, semA)
            fill = scan_chunk(srcA, dstA, fill)
            stage((k0 + 2) % NCH, srcA, dstA, semA)
            wait_stage(srcB, dstB, semB)
            fill = scan_chunk(srcB, dstB, fill)
            return fill

        fill = lax.fori_loop(0, NCH // 2, pair, jnp.int32(0))
        wait_stage(srcA, dstA, semA)

        zi = jnp.zeros((16,), jnp.int32)
        nbv = jnp.full((16,), NB, jnp.int32)
        for q in range(WG):
            csrc[pl.ds(fill + 16 * q, 16)] = zi
            cdst[pl.ds(fill + 16 * q, 16)] = nbv
        do_wave(0)

        def rdiv(r, _):
            dv = den_v[pl.ds(r * HH, 16)]
            rv = 1.0 / (dv + 1e-9)
            for h in range(HH):
                rsc = rv[h]
                for cc in range(CPH):
                    c = h * CPH + cc
                    acc[r, pl.ds(16 * c, 16)] = acc[r, pl.ds(16 * c, 16)] * rsc
            return 0
        lax.fori_loop(0, NB, rdiv, 0)
        for t in range(NB // 64):
            pltpu.sync_copy(acc.at[pl.ds(t * 64, 64)],
                            out_h.at[pl.ds(base + t * 64, 64)])

    return kern(edge, tab, el, er)


def kernel(x, edge_index, W1, al1, ar1, W2, al2, ar2):
    xp = jnp.zeros((NA, D), jnp.float32).at[:N].set(x)
    fA, fB, elA, erA, elB, erB = _tc_layer1(xp, W1, al1, ar1)
    o1A = _sc_pass(256, 2, edge_index, fA, elA.reshape(-1), erA.reshape(-1))
    o1B = _sc_pass(256, 2, edge_index, fB, elB.reshape(-1), erB.reshape(-1))
    f2, el2, er2 = _tc_layer2(o1A, o1B, W2, al2, ar2)
    o2 = _sc_pass(128, 1, edge_index, f2, el2.reshape(-1), er2.reshape(-1))
    return o2[:N]


# speedup vs baseline: 9.7213x; 1.0617x over previous
import dataclasses
import functools

import jax
import jax.numpy as jnp
from jax import lax
from jax.experimental import pallas as pl
from jax.experimental.pallas import tpu as pltpu
from jax.experimental.pallas import tpu_sc as plsc

N = 10000
E = 320000
D = 128
NEG = 0.2

NW = 32
NB = 320
NA = 10496
NE_T = 10016
C = 1280
NCH = E // C
G = C // 16
W = 64

_CP = pltpu.CompilerParams()
if "needs_layout_passes" in pltpu.CompilerParams.__dataclass_fields__:
    _CP = dataclasses.replace(_CP, needs_layout_passes=False)

_MESH = plsc.VectorSubcoreMesh(core_axis_name="c", subcore_axis_name="s")


def _tc_layer1(xp, W1, al1, ar1):
    BR = 256
    grid = (NA // BR,)

    def body(x_ref, w_ref, al_ref, ar_ref, fA, fB, elA, erA, elB, erB):
        feat = jnp.dot(x_ref[...], w_ref[...], preferred_element_type=jnp.float32)
        fA[...] = feat[:, :256]
        fB[...] = feat[:, 256:]
        f4 = feat.reshape(BR, 4, 128)
        el = jnp.sum(f4 * al_ref[...][None], axis=-1)
        er = jnp.sum(f4 * ar_ref[...][None], axis=-1)
        elA[...] = el[:, 0:2]
        erA[...] = er[:, 0:2]
        elB[...] = el[:, 2:4]
        erB[...] = er[:, 2:4]

    f32 = jnp.float32
    return pl.pallas_call(
        body,
        grid=grid,
        in_specs=[
            pl.BlockSpec((BR, D), lambda i: (i, 0)),
            pl.BlockSpec((D, 512), lambda i: (0, 0)),
            pl.BlockSpec((4, 128), lambda i: (0, 0)),
            pl.BlockSpec((4, 128), lambda i: (0, 0)),
        ],
        out_specs=[
            pl.BlockSpec((BR, 256), lambda i: (i, 0)),
            pl.BlockSpec((BR, 256), lambda i: (i, 0)),
            pl.BlockSpec((BR, 2), lambda i: (i, 0)),
            pl.BlockSpec((BR, 2), lambda i: (i, 0)),
            pl.BlockSpec((BR, 2), lambda i: (i, 0)),
            pl.BlockSpec((BR, 2), lambda i: (i, 0)),
        ],
        out_shape=[
            jax.ShapeDtypeStruct((NA, 256), f32),
            jax.ShapeDtypeStruct((NA, 256), f32),
            jax.ShapeDtypeStruct((NA, 2), f32),
            jax.ShapeDtypeStruct((NA, 2), f32),
            jax.ShapeDtypeStruct((NA, 2), f32),
            jax.ShapeDtypeStruct((NA, 2), f32),
        ],
    )(xp, W1, al1, ar1)


def _tc_layer2(o1A, o1B, W2, al2, ar2):
    BR = 256
    grid = (NA // BR,)

    def body(a_ref, b_ref, w_ref, al_ref, ar_ref, f2, el2, er2):
        a = a_ref[...]
        b = b_ref[...]

        def elu(v):
            return jnp.where(v > 0, v, jnp.exp(v) - 1.0)

        hmean = 0.25 * (elu(a[:, :128]) + elu(a[:, 128:])
                        + elu(b[:, :128]) + elu(b[:, 128:]))
        f = jnp.dot(hmean, w_ref[...], preferred_element_type=jnp.float32)
        f2[...] = f
        el2[...] = jnp.sum(f * al_ref[...], axis=-1, keepdims=True)
        er2[...] = jnp.sum(f * ar_ref[...], axis=-1, keepdims=True)

    f32 = jnp.float32
    return pl.pallas_call(
        body,
        grid=grid,
        in_specs=[
            pl.BlockSpec((BR, 256), lambda i: (i, 0)),
            pl.BlockSpec((BR, 256), lambda i: (i, 0)),
            pl.BlockSpec((128, 128), lambda i: (0, 0)),
            pl.BlockSpec((1, 128), lambda i: (0, 0)),
            pl.BlockSpec((1, 128), lambda i: (0, 0)),
        ],
        out_specs=[
            pl.BlockSpec((BR, 128), lambda i: (i, 0)),
            pl.BlockSpec((BR, 1), lambda i: (i, 0)),
            pl.BlockSpec((BR, 1), lambda i: (i, 0)),
        ],
        out_shape=[
            jax.ShapeDtypeStruct((NA, 128), f32),
            jax.ShapeDtypeStruct((NA, 1), f32),
            jax.ShapeDtypeStruct((NA, 1), f32),
        ],
    )(o1A, o1B, W2, al2, ar2)


def _sc_pass(TW, HH, edge, tab, el, er):
    CPH = 128 // 16
    NC_ = TW // 16
    WG = W // 16

    @functools.partial(
        pl.kernel,
        out_type=jax.ShapeDtypeStruct((NA, TW), jnp.float32),
        mesh=_MESH,
        scratch_types=[
            pltpu.VMEM((NB + 1, TW), jnp.float32),
            pltpu.VMEM((NE_T * HH,), jnp.float32),
            pltpu.VMEM((336 * HH,), jnp.float32),
            pltpu.VMEM((352 * HH,), jnp.float32),
            pltpu.VMEM((C,), jnp.int32),
            pltpu.VMEM((C,), jnp.int32),
            pltpu.VMEM((C,), jnp.int32),
            pltpu.VMEM((C,), jnp.int32),
            pltpu.VMEM((C + 96,), jnp.int32),
            pltpu.VMEM((C + 96,), jnp.int32),
            pltpu.VMEM((W, TW), jnp.float32),
            pltpu.VMEM((2 * W + 32,), jnp.float32),
            pltpu.SemaphoreType.DMA,
            pltpu.SemaphoreType.DMA,
            pltpu.SemaphoreType.DMA,
        ],
        compiler_params=_CP,
    )
    def kern(edge_h, tab_h, el_h, er_h, out_h,
             acc, el_v, er_v, den_v, srcA, dstA, srcB, dstB, csrc, cdst,
             rows, eebuf, semg, semA, semB):
        cid = lax.axis_index("c")
        sid = lax.axis_index("s")
        wid = sid * 2 + cid
        base = wid * NB
        z16 = jnp.zeros((16,), jnp.float32)

        for q in range(352 * HH // 16):
            den_v[pl.ds(16 * q, 16)] = z16

        def zr(j, _):
            for cc in range(NC_):
                acc[j, pl.ds(16 * cc, 16)] = z16
            return 0
        lax.fori_loop(0, NB + 1, zr, 0)

        pltpu.sync_copy(el_h.at[pl.ds(0, NE_T * HH)], el_v)
        pltpu.sync_copy(er_h.at[pl.ds(base * HH, 336 * HH)], er_v)

        def do_wave(off):
            gcp = pltpu.async_copy(tab_h.at[csrc.at[pl.ds(off, W)]], rows, semg)
            for h in range(HH):
                for q in range(WG):
                    sv = csrc[pl.ds(off + 16 * q, 16)]
                    rl = cdst[pl.ds(off + 16 * q, 16)]
                    elh = plsc.load_gather(el_v, [sv * HH + h])
                    erh = plsc.load_gather(er_v, [rl * HH + h])
                    e = elh + erh
                    e = jnp.maximum(e, NEG * e)
                    ee = jnp.exp(e)
                    plsc.addupdate_scatter(den_v, [rl * HH + h], ee)
                    eebuf[pl.ds(h * W + 16 * q, 16)] = ee
            gcp.wait()

            def accj(j, _):
                rl = cdst[pl.ds(off + j, 16)][0]
                for h in range(HH):
                    eesc = eebuf[pl.ds(h * W + j, 16)][0]
                    for cc in range(CPH):
                        c = h * CPH + cc
                        val = rows[j, pl.ds(16 * c, 16)] * eesc
                        plsc.addupdate(acc.at[rl, pl.ds(16 * c, 16)], val)
                return 0
            lax.fori_loop(0, W, accj, 0)

        def scan_chunk(sbuf, dbuf, fill):
            def grp(g, fl):
                dvec = dbuf[pl.ds(16 * g, 16)]
                svec = sbuf[pl.ds(16 * g, 16)]
                rel = dvec - base
                mask = rel.astype(jnp.uint32) < jnp.uint32(NB)
                plsc.store_compressed(csrc.at[pl.ds(fl, 16)], svec, mask=mask)
                plsc.store_compressed(cdst.at[pl.ds(fl, 16)], rel, mask=mask)
                cnt = plsc.all_reduce_population_count(mask)
                return fl + (cnt if cnt.ndim == 0 else cnt[0])

            fill = lax.fori_loop(0, G, grp, fill)
            nw = fill // W

            def wv(w, _):
                do_wave(w * W)
                return 0
            lax.fori_loop(0, nw, wv, 0)
            tail = nw * W
            for q in range(WG):
                sv = csrc[pl.ds(tail + 16 * q, 16)]
                rv = cdst[pl.ds(tail + 16 * q, 16)]
                csrc[pl.ds(16 * q, 16)] = sv
                cdst[pl.ds(16 * q, 16)] = rv
            return fill - tail

        def stage(k, sbuf, dbuf, sem):
            pltpu.async_copy(edge_h.at[0, pl.ds(k * C, C)], sbuf, sem)
            pltpu.async_copy(edge_h.at[1, pl.ds(k * C, C)], dbuf, sem)

        def wait_stage(sbuf, dbuf, sem):
            pltpu.make_async_copy(edge_h.at[0, pl.ds(0, C)], sbuf, sem).wait()
            pltpu.make_async_copy(edge_h.at[1, pl.ds(0, C)], dbuf, sem).wait()

        stage(0, srcA, dstA, semA)

        def pair(i, fill):
            k0 = 2 * i
            stage(k0 + 1, srcB, dstB, semB)
            wait_stage(srcA, dstA, semA)
            fill = scan_chunk(srcA, dstA, fill)
            stage((k0 + 2) % NCH, srcA, dstA, semA)
            wait_stage(srcB, dstB, semB)
            fill = scan_chunk(srcB, dstB, fill)
            return fill

        fill = lax.fori_loop(0, NCH // 2, pair, jnp.int32(0))
        wait_stage(srcA, dstA, semA)

        zi = jnp.zeros((16,), jnp.int32)
        nbv = jnp.full((16,), NB, jnp.int32)
        for q in range(WG):
            csrc[pl.ds(fill + 16 * q, 16)] = zi
            cdst[pl.ds(fill + 16 * q, 16)] = nbv
        do_wave(0)

        def rdiv(r, _):
            dv = den_v[pl.ds(r * HH, 16)]
            rv = 1.0 / (dv + 1e-9)
            for h in range(HH):
                rsc = rv[h]
                for cc in range(CPH):
                    c = h * CPH + cc
                    acc[r, pl.ds(16 * c, 16)] = acc[r, pl.ds(16 * c, 16)] * rsc
            return 0
        lax.fori_loop(0, NB, rdiv, 0)
        for t in range(NB // 64):
            pltpu.sync_copy(acc.at[pl.ds(t * 64, 64)],
                            out_h.at[pl.ds(base + t * 64, 64)])

    return kern(edge, tab, el, er)


def kernel(x, edge_index, W1, al1, ar1, W2, al2, ar2):
    xp = jnp.zeros((NA, D), jnp.float32).at[:N].set(x)
    fA, fB, elA, erA, elB, erB = _tc_layer1(xp, W1, al1, ar1)
    o1A = _sc_pass(256, 2, edge_index, fA, elA.reshape(-1), erA.reshape(-1))
    o1B = _sc_pass(256, 2, edge_index, fB, elB.reshape(-1), erB.reshape(-1))
    f2, el2, er2 = _tc_layer2(o1A, o1B, W2, al2, ar2)
    o2 = _sc_pass(128, 1, edge_index, f2, el2.reshape(-1), er2.reshape(-1))
    return o2[:N]


# speedup vs baseline: 10.9334x; 1.1247x over previous
import dataclasses
import functools

import jax
import jax.numpy as jnp
from jax import lax
from jax.experimental import pallas as pl
from jax.experimental.pallas import tpu as pltpu
from jax.experimental.pallas import tpu_sc as plsc

N = 10000
E = 320000
D = 128
NEG = 0.2

NW = 32
NB = 320
NA = 10496
NE_T = 10016
C = 1280
NCH = E // C
G = C // 16
W = 64

_CP = pltpu.CompilerParams()
if "needs_layout_passes" in pltpu.CompilerParams.__dataclass_fields__:
    _CP = dataclasses.replace(_CP, needs_layout_passes=False)

_MESH = plsc.VectorSubcoreMesh(core_axis_name="c", subcore_axis_name="s")


def _tc_layer1(xp, W1, al1, ar1):
    BR = 256
    grid = (NA // BR,)

    def body(x_ref, w_ref, al_ref, ar_ref, fA, fB, elA, erA, elB, erB):
        feat = jnp.dot(x_ref[...], w_ref[...], preferred_element_type=jnp.float32)
        fA[...] = feat[:, :256]
        fB[...] = feat[:, 256:]
        f4 = feat.reshape(BR, 4, 128)
        el = jnp.sum(f4 * al_ref[...][None], axis=-1)
        er = jnp.sum(f4 * ar_ref[...][None], axis=-1)
        elA[...] = el[:, 0:2]
        erA[...] = er[:, 0:2]
        elB[...] = el[:, 2:4]
        erB[...] = er[:, 2:4]

    f32 = jnp.float32
    return pl.pallas_call(
        body,
        grid=grid,
        in_specs=[
            pl.BlockSpec((BR, D), lambda i: (i, 0)),
            pl.BlockSpec((D, 512), lambda i: (0, 0)),
            pl.BlockSpec((4, 128), lambda i: (0, 0)),
            pl.BlockSpec((4, 128), lambda i: (0, 0)),
        ],
        out_specs=[
            pl.BlockSpec((BR, 256), lambda i: (i, 0)),
            pl.BlockSpec((BR, 256), lambda i: (i, 0)),
            pl.BlockSpec((BR, 2), lambda i: (i, 0)),
            pl.BlockSpec((BR, 2), lambda i: (i, 0)),
            pl.BlockSpec((BR, 2), lambda i: (i, 0)),
            pl.BlockSpec((BR, 2), lambda i: (i, 0)),
        ],
        out_shape=[
            jax.ShapeDtypeStruct((NA, 256), f32),
            jax.ShapeDtypeStruct((NA, 256), f32),
            jax.ShapeDtypeStruct((NA, 2), f32),
            jax.ShapeDtypeStruct((NA, 2), f32),
            jax.ShapeDtypeStruct((NA, 2), f32),
            jax.ShapeDtypeStruct((NA, 2), f32),
        ],
    )(xp, W1, al1, ar1)


def _tc_layer2(o1A, o1B, W2, al2, ar2):
    BR = 256
    grid = (NA // BR,)

    def body(a_ref, b_ref, w_ref, al_ref, ar_ref, f2, el2, er2):
        a = a_ref[...]
        b = b_ref[...]

        def elu(v):
            return jnp.where(v > 0, v, jnp.exp(v) - 1.0)

        hmean = 0.25 * (elu(a[:, :128]) + elu(a[:, 128:])
                        + elu(b[:, :128]) + elu(b[:, 128:]))
        f = jnp.dot(hmean, w_ref[...], preferred_element_type=jnp.float32)
        f2[...] = f
        el2[...] = jnp.sum(f * al_ref[...], axis=-1, keepdims=True)
        er2[...] = jnp.sum(f * ar_ref[...], axis=-1, keepdims=True)

    f32 = jnp.float32
    return pl.pallas_call(
        body,
        grid=grid,
        in_specs=[
            pl.BlockSpec((BR, 256), lambda i: (i, 0)),
            pl.BlockSpec((BR, 256), lambda i: (i, 0)),
            pl.BlockSpec((128, 128), lambda i: (0, 0)),
            pl.BlockSpec((1, 128), lambda i: (0, 0)),
            pl.BlockSpec((1, 128), lambda i: (0, 0)),
        ],
        out_specs=[
            pl.BlockSpec((BR, 128), lambda i: (i, 0)),
            pl.BlockSpec((BR, 1), lambda i: (i, 0)),
            pl.BlockSpec((BR, 1), lambda i: (i, 0)),
        ],
        out_shape=[
            jax.ShapeDtypeStruct((NA, 128), f32),
            jax.ShapeDtypeStruct((NA, 1), f32),
            jax.ShapeDtypeStruct((NA, 1), f32),
        ],
    )(o1A, o1B, W2, al2, ar2)


def _sc_pass(TW, HH, edge, tab, el, er):
    CPH = 128 // 16
    NC_ = TW // 16
    WG = W // 16

    @functools.partial(
        pl.kernel,
        out_type=jax.ShapeDtypeStruct((NA, TW), jnp.float32),
        mesh=_MESH,
        scratch_types=[
            pltpu.VMEM((NB + 1, TW), jnp.float32),
            pltpu.VMEM((NE_T * HH,), jnp.float32),
            pltpu.VMEM((336 * HH,), jnp.float32),
            pltpu.VMEM((352 * HH,), jnp.float32),
            pltpu.VMEM((C,), jnp.int32),
            pltpu.VMEM((C,), jnp.int32),
            pltpu.VMEM((C,), jnp.int32),
            pltpu.VMEM((C,), jnp.int32),
            pltpu.VMEM((C + 96,), jnp.int32),
            pltpu.VMEM((C + 96,), jnp.int32),
            pltpu.VMEM((W, TW), jnp.float32),
            pltpu.VMEM((2 * W + 32,), jnp.float32),
            pltpu.VMEM((W + 16,), jnp.int32),
            pltpu.VMEM((W,), jnp.int32),
            pltpu.SemaphoreType.DMA,
            pltpu.SemaphoreType.DMA,
            pltpu.SemaphoreType.DMA,
        ],
        compiler_params=_CP,
    )
    def kern(edge_h, tab_h, el_h, er_h, out_h,
             acc, el_v, er_v, den_v, srcA, dstA, srcB, dstB, csrc, cdst,
             rows, eebuf, pendrl, wavesrc, semg, semA, semB):
        cid = lax.axis_index("c")
        sid = lax.axis_index("s")
        wid = sid * 2 + cid
        base = wid * NB
        z16 = jnp.zeros((16,), jnp.float32)

        for q in range(352 * HH // 16):
            den_v[pl.ds(16 * q, 16)] = z16

        def zr(j, _):
            for cc in range(NC_):
                acc[j, pl.ds(16 * cc, 16)] = z16
            return 0
        lax.fori_loop(0, NB + 1, zr, 0)

        pltpu.sync_copy(el_h.at[pl.ds(0, NE_T * HH)], el_v)
        pltpu.sync_copy(er_h.at[pl.ds(base * HH, 336 * HH)], er_v)

        def issue_wave(off):
            for q in range(WG):
                sv = csrc[pl.ds(off + 16 * q, 16)]
                rl = cdst[pl.ds(off + 16 * q, 16)]
                wavesrc[pl.ds(16 * q, 16)] = sv
                pendrl[pl.ds(16 * q, 16)] = rl
                for h in range(HH):
                    elh = plsc.load_gather(el_v, [sv * HH + h])
                    erh = plsc.load_gather(er_v, [rl * HH + h])
                    e = elh + erh
                    e = jnp.maximum(e, NEG * e)
                    ee = jnp.exp(e)
                    plsc.addupdate_scatter(den_v, [rl * HH + h], ee)
                    eebuf[pl.ds(h * W + 16 * q, 16)] = ee
            pltpu.async_copy(tab_h.at[wavesrc], rows, semg)

        def proc_pending():
            pltpu.make_async_copy(tab_h.at[wavesrc], rows, semg).wait()

            def accj(j, _):
                rl = pendrl[pl.ds(j, 16)][0]
                for h in range(HH):
                    eesc = eebuf[pl.ds(h * W + j, 16)][0]
                    for cc in range(CPH):
                        c = h * CPH + cc
                        val = rows[j, pl.ds(16 * c, 16)] * eesc
                        plsc.addupdate(acc.at[rl, pl.ds(16 * c, 16)], val)
                return 0
            lax.fori_loop(0, W, accj, 0)

        def scan_chunk(sbuf, dbuf, carry):
            fill, pend = carry

            def grp(g, fl):
                dvec = dbuf[pl.ds(16 * g, 16)]
                svec = sbuf[pl.ds(16 * g, 16)]
                rel = dvec - base
                mask = rel.astype(jnp.uint32) < jnp.uint32(NB)
                plsc.store_compressed(csrc.at[pl.ds(fl, 16)], svec, mask=mask)
                plsc.store_compressed(cdst.at[pl.ds(fl, 16)], rel, mask=mask)
                cnt = plsc.all_reduce_population_count(mask)
                return fl + (cnt if cnt.ndim == 0 else cnt[0])

            fill = lax.fori_loop(0, G, grp, fill)
            nw = fill // W

            def wv(w, pnd):
                @pl.when(pnd == 1)
                def _():
                    proc_pending()
                issue_wave(w * W)
                return jnp.int32(1)
            pend = lax.fori_loop(0, nw, wv, pend)
            tail = nw * W
            for q in range(WG):
                sv = csrc[pl.ds(tail + 16 * q, 16)]
                rv = cdst[pl.ds(tail + 16 * q, 16)]
                csrc[pl.ds(16 * q, 16)] = sv
                cdst[pl.ds(16 * q, 16)] = rv
            return fill - tail, pend

        def stage(k, sbuf, dbuf, sem):
            pltpu.async_copy(edge_h.at[0, pl.ds(k * C, C)], sbuf, sem)
            pltpu.async_copy(edge_h.at[1, pl.ds(k * C, C)], dbuf, sem)

        def wait_stage(sbuf, dbuf, sem):
            pltpu.make_async_copy(edge_h.at[0, pl.ds(0, C)], sbuf, sem).wait()
            pltpu.make_async_copy(edge_h.at[1, pl.ds(0, C)], dbuf, sem).wait()

        stage(0, srcA, dstA, semA)

        def pair(i, carry):
            k0 = 2 * i
            stage(k0 + 1, srcB, dstB, semB)
            wait_stage(srcA, dstA, semA)
            carry = scan_chunk(srcA, dstA, carry)
            stage((k0 + 2) % NCH, srcA, dstA, semA)
            wait_stage(srcB, dstB, semB)
            carry = scan_chunk(srcB, dstB, carry)
            return carry

        fill, pend = lax.fori_loop(0, NCH // 2, pair,
                                   (jnp.int32(0), jnp.int32(0)))
        wait_stage(srcA, dstA, semA)

        @pl.when(pend == 1)
        def _():
            proc_pending()

        zi = jnp.zeros((16,), jnp.int32)
        nbv = jnp.full((16,), NB, jnp.int32)
        for q in range(WG):
            csrc[pl.ds(fill + 16 * q, 16)] = zi
            cdst[pl.ds(fill + 16 * q, 16)] = nbv
        issue_wave(0)
        proc_pending()

        def rdiv(r, _):
            dv = den_v[pl.ds(r * HH, 16)]
            rv = 1.0 / (dv + 1e-9)
            for h in range(HH):
                rsc = rv[h]
                for cc in range(CPH):
                    c = h * CPH + cc
                    acc[r, pl.ds(16 * c, 16)] = acc[r, pl.ds(16 * c, 16)] * rsc
            return 0
        lax.fori_loop(0, NB, rdiv, 0)
        for t in range(NB // 64):
            pltpu.sync_copy(acc.at[pl.ds(t * 64, 64)],
                            out_h.at[pl.ds(base + t * 64, 64)])

    return kern(edge, tab, el, er)


def kernel(x, edge_index, W1, al1, ar1, W2, al2, ar2):
    xp = jnp.zeros((NA, D), jnp.float32).at[:N].set(x)
    fA, fB, elA, erA, elB, erB = _tc_layer1(xp, W1, al1, ar1)
    o1A = _sc_pass(256, 2, edge_index, fA, elA.reshape(-1), erA.reshape(-1))
    o1B = _sc_pass(256, 2, edge_index, fB, elB.reshape(-1), erB.reshape(-1))
    f2, el2, er2 = _tc_layer2(o1A, o1B, W2, al2, ar2)
    o2 = _sc_pass(128, 1, edge_index, f2, el2.reshape(-1), er2.reshape(-1))
    return o2[:N]


# speedup vs baseline: 11.1826x; 1.0228x over previous
import dataclasses
import functools

import jax
import jax.numpy as jnp
from jax import lax
from jax.experimental import pallas as pl
from jax.experimental.pallas import tpu as pltpu
from jax.experimental.pallas import tpu_sc as plsc

N = 10000
E = 320000
D = 128
NEG = 0.2

NW = 32
NB = 320
NA = 10496
NE_T = 10016
C = 1280
NCH = E // C
G = C // 16
W = 64

_CP = pltpu.CompilerParams()
if "needs_layout_passes" in pltpu.CompilerParams.__dataclass_fields__:
    _CP = dataclasses.replace(_CP, needs_layout_passes=False)

_MESH = plsc.VectorSubcoreMesh(core_axis_name="c", subcore_axis_name="s")


def _tc_layer1(xp, W1, al1, ar1):
    BR = 256
    grid = (NA // BR,)

    def body(x_ref, w_ref, al_ref, ar_ref, fA, fB, elA, erA, elB, erB):
        feat = jnp.dot(x_ref[...], w_ref[...], preferred_element_type=jnp.float32)
        fA[...] = feat[:, :256]
        fB[...] = feat[:, 256:]
        f4 = feat.reshape(BR, 4, 128)
        el = jnp.sum(f4 * al_ref[...][None], axis=-1)
        er = jnp.sum(f4 * ar_ref[...][None], axis=-1)
        elA[...] = el[:, 0:2]
        erA[...] = er[:, 0:2]
        elB[...] = el[:, 2:4]
        erB[...] = er[:, 2:4]

    f32 = jnp.float32
    return pl.pallas_call(
        body,
        grid=grid,
        in_specs=[
            pl.BlockSpec((BR, D), lambda i: (i, 0)),
            pl.BlockSpec((D, 512), lambda i: (0, 0)),
            pl.BlockSpec((4, 128), lambda i: (0, 0)),
            pl.BlockSpec((4, 128), lambda i: (0, 0)),
        ],
        out_specs=[
            pl.BlockSpec((BR, 256), lambda i: (i, 0)),
            pl.BlockSpec((BR, 256), lambda i: (i, 0)),
            pl.BlockSpec((BR, 2), lambda i: (i, 0)),
            pl.BlockSpec((BR, 2), lambda i: (i, 0)),
            pl.BlockSpec((BR, 2), lambda i: (i, 0)),
            pl.BlockSpec((BR, 2), lambda i: (i, 0)),
        ],
        out_shape=[
            jax.ShapeDtypeStruct((NA, 256), f32),
            jax.ShapeDtypeStruct((NA, 256), f32),
            jax.ShapeDtypeStruct((NA, 2), f32),
            jax.ShapeDtypeStruct((NA, 2), f32),
            jax.ShapeDtypeStruct((NA, 2), f32),
            jax.ShapeDtypeStruct((NA, 2), f32),
        ],
    )(xp, W1, al1, ar1)


def _tc_layer2(o1A, o1B, W2, al2, ar2):
    BR = 256
    grid = (NA // BR,)

    def body(a_ref, b_ref, w_ref, al_ref, ar_ref, f2, el2, er2):
        a = a_ref[...]
        b = b_ref[...]

        def elu(v):
            return jnp.where(v > 0, v, jnp.exp(v) - 1.0)

        hmean = 0.25 * (elu(a[:, :128]) + elu(a[:, 128:])
                        + elu(b[:, :128]) + elu(b[:, 128:]))
        f = jnp.dot(hmean, w_ref[...], preferred_element_type=jnp.float32)
        f2[...] = f
        el2[...] = jnp.sum(f * al_ref[...], axis=-1, keepdims=True)
        er2[...] = jnp.sum(f * ar_ref[...], axis=-1, keepdims=True)

    f32 = jnp.float32
    return pl.pallas_call(
        body,
        grid=grid,
        in_specs=[
            pl.BlockSpec((BR, 256), lambda i: (i, 0)),
            pl.BlockSpec((BR, 256), lambda i: (i, 0)),
            pl.BlockSpec((128, 128), lambda i: (0, 0)),
            pl.BlockSpec((1, 128), lambda i: (0, 0)),
            pl.BlockSpec((1, 128), lambda i: (0, 0)),
        ],
        out_specs=[
            pl.BlockSpec((BR, 128), lambda i: (i, 0)),
            pl.BlockSpec((BR, 1), lambda i: (i, 0)),
            pl.BlockSpec((BR, 1), lambda i: (i, 0)),
        ],
        out_shape=[
            jax.ShapeDtypeStruct((NA, 128), f32),
            jax.ShapeDtypeStruct((NA, 1), f32),
            jax.ShapeDtypeStruct((NA, 1), f32),
        ],
    )(o1A, o1B, W2, al2, ar2)


def _sc_pass(TW, HH, edge, tab, el, er):
    CPH = 128 // 16
    NC_ = TW // 16
    WG = W // 16

    @functools.partial(
        pl.kernel,
        out_type=jax.ShapeDtypeStruct((NA, TW), jnp.float32),
        mesh=_MESH,
        scratch_types=[
            pltpu.VMEM((NB + 1, TW), jnp.float32),
            pltpu.VMEM((NE_T * HH,), jnp.float32),
            pltpu.VMEM((336 * HH,), jnp.float32),
            pltpu.VMEM((352 * HH,), jnp.float32),
            pltpu.VMEM((C,), jnp.int32),
            pltpu.VMEM((C,), jnp.int32),
            pltpu.VMEM((C,), jnp.int32),
            pltpu.VMEM((C,), jnp.int32),
            pltpu.VMEM((C + 96,), jnp.int32),
            pltpu.VMEM((C + 96,), jnp.int32),
            pltpu.VMEM((W, TW), jnp.float32),
            pltpu.VMEM((2 * W + 32,), jnp.float32),
            pltpu.VMEM((W + 16,), jnp.int32),
            pltpu.VMEM((W,), jnp.int32),
            pltpu.SemaphoreType.DMA,
            pltpu.SemaphoreType.DMA,
            pltpu.SemaphoreType.DMA,
        ],
        compiler_params=_CP,
    )
    def kern(edge_h, tab_h, el_h, er_h, out_h,
             acc, el_v, er_v, den_v, srcA, dstA, srcB, dstB, csrc, cdst,
             rows, eebuf, pendrl, wavesrc, semg, semA, semB):
        cid = lax.axis_index("c")
        sid = lax.axis_index("s")
        wid = sid * 2 + cid
        base = wid * NB
        z16 = jnp.zeros((16,), jnp.float32)

        for q in range(352 * HH // 16):
            den_v[pl.ds(16 * q, 16)] = z16

        def zr(j, _):
            for cc in range(NC_):
                acc[j, pl.ds(16 * cc, 16)] = z16
            return 0
        lax.fori_loop(0, NB + 1, zr, 0)

        pltpu.sync_copy(el_h.at[pl.ds(0, NE_T * HH)], el_v)
        pltpu.sync_copy(er_h.at[pl.ds(base * HH, 336 * HH)], er_v)

        def issue_wave(off):
            for q in range(WG):
                sv = csrc[pl.ds(off + 16 * q, 16)]
                rl = cdst[pl.ds(off + 16 * q, 16)]
                wavesrc[pl.ds(16 * q, 16)] = sv
                pendrl[pl.ds(16 * q, 16)] = rl
                for h in range(HH):
                    elh = plsc.load_gather(el_v, [sv * HH + h])
                    erh = plsc.load_gather(er_v, [rl * HH + h])
                    e = elh + erh
                    e = jnp.maximum(e, NEG * e)
                    ee = jnp.exp(e)
                    plsc.addupdate_scatter(den_v, [rl * HH + h], ee)
                    eebuf[pl.ds(h * W + 16 * q, 16)] = ee
            pltpu.async_copy(tab_h.at[wavesrc], rows, semg)

        def proc_pending():
            pltpu.make_async_copy(tab_h.at[wavesrc], rows, semg).wait()

            def accj(j2, _):
                for u in range(2):
                    j = 2 * j2 + u
                    rl = pendrl[pl.ds(j, 16)][0]
                    for h in range(HH):
                        eesc = eebuf[pl.ds(h * W + j, 16)][0]
                        for cc in range(CPH):
                            c = h * CPH + cc
                            val = rows[j, pl.ds(16 * c, 16)] * eesc
                            plsc.addupdate(acc.at[rl, pl.ds(16 * c, 16)], val)
                return 0
            lax.fori_loop(0, W // 2, accj, 0)

        def scan_chunk(sbuf, dbuf, carry):
            fill, pend = carry

            def grp(g, fl):
                for u in range(4):
                    gg = 4 * g + u
                    dvec = dbuf[pl.ds(16 * gg, 16)]
                    svec = sbuf[pl.ds(16 * gg, 16)]
                    rel = dvec - base
                    mask = rel.astype(jnp.uint32) < jnp.uint32(NB)
                    plsc.store_compressed(csrc.at[pl.ds(fl, 16)], svec, mask=mask)
                    plsc.store_compressed(cdst.at[pl.ds(fl, 16)], rel, mask=mask)
                    cnt = plsc.all_reduce_population_count(mask)
                    fl = fl + (cnt if cnt.ndim == 0 else cnt[0])
                return fl

            fill = lax.fori_loop(0, G // 4, grp, fill)
            nw = fill // W

            def wv(w, pnd):
                @pl.when(pnd == 1)
                def _():
                    proc_pending()
                issue_wave(w * W)
                return jnp.int32(1)
            pend = lax.fori_loop(0, nw, wv, pend)
            tail = nw * W
            for q in range(WG):
                sv = csrc[pl.ds(tail + 16 * q, 16)]
                rv = cdst[pl.ds(tail + 16 * q, 16)]
                csrc[pl.ds(16 * q, 16)] = sv
                cdst[pl.ds(16 * q, 16)] = rv
            return fill - tail, pend

        def stage(k, sbuf, dbuf, sem):
            pltpu.async_copy(edge_h.at[0, pl.ds(k * C, C)], sbuf, sem)
            pltpu.async_copy(edge_h.at[1, pl.ds(k * C, C)], dbuf, sem)

        def wait_stage(sbuf, dbuf, sem):
            pltpu.make_async_copy(edge_h.at[0, pl.ds(0, C)], sbuf, sem).wait()
            pltpu.make_async_copy(edge_h.at[1, pl.ds(0, C)], dbuf, sem).wait()

        stage(0, srcA, dstA, semA)

        def pair(i, carry):
            k0 = 2 * i
            stage(k0 + 1, srcB, dstB, semB)
            wait_stage(srcA, dstA, semA)
            carry = scan_chunk(srcA, dstA, carry)
            stage((k0 + 2) % NCH, srcA, dstA, semA)
            wait_stage(srcB, dstB, semB)
            carry = scan_chunk(srcB, dstB, carry)
            return carry

        fill, pend = lax.fori_loop(0, NCH // 2, pair,
                                   (jnp.int32(0), jnp.int32(0)))
        wait_stage(srcA, dstA, semA)

        @pl.when(pend == 1)
        def _():
            proc_pending()

        zi = jnp.zeros((16,), jnp.int32)
        nbv = jnp.full((16,), NB, jnp.int32)
        for q in range(WG):
            csrc[pl.ds(fill + 16 * q, 16)] = zi
            cdst[pl.ds(fill + 16 * q, 16)] = nbv
        issue_wave(0)
        proc_pending()

        def rdiv(r, _):
            dv = den_v[pl.ds(r * HH, 16)]
            rv = 1.0 / (dv + 1e-9)
            for h in range(HH):
                rsc = rv[h]
                for cc in range(CPH):
                    c = h * CPH + cc
                    acc[r, pl.ds(16 * c, 16)] = acc[r, pl.ds(16 * c, 16)] * rsc
            return 0
        lax.fori_loop(0, NB, rdiv, 0)
        for t in range(NB // 64):
            pltpu.sync_copy(acc.at[pl.ds(t * 64, 64)],
                            out_h.at[pl.ds(base + t * 64, 64)])

    return kern(edge, tab, el, er)


def kernel(x, edge_index, W1, al1, ar1, W2, al2, ar2):
    xp = jnp.zeros((NA, D), jnp.float32).at[:N].set(x)
    fA, fB, elA, erA, elB, erB = _tc_layer1(xp, W1, al1, ar1)
    o1A = _sc_pass(256, 2, edge_index, fA, elA.reshape(-1), erA.reshape(-1))
    o1B = _sc_pass(256, 2, edge_index, fB, elB.reshape(-1), erB.reshape(-1))
    f2, el2, er2 = _tc_layer2(o1A, o1B, W2, al2, ar2)
    o2 = _sc_pass(128, 1, edge_index, f2, el2.reshape(-1), er2.reshape(-1))
    return o2[:N]
